# baseline (speedup 1.0000x reference)
import functools

import jax
import jax.numpy as jnp
from jax import lax
from jax.experimental import pallas as pl
from jax.experimental.pallas import tpu as pltpu
from jax.experimental.pallas import tpu_sc as plsc

_LN_EPS = 1e-12
_GATHER_WINDOW = 128
_NUM_CHUNKS = 1
_BT = 4096


def _sc_gather(table, idx2d):
    nwin, w = idx2d.shape
    n = nwin * w
    h = table.shape[1]
    nworkers = 32
    wpw = nwin // nworkers
    half = wpw // 2
    mesh = plsc.VectorSubcoreMesh(core_axis_name="core", subcore_axis_name="subcore")

    @functools.partial(
        pl.kernel,
        out_type=jax.ShapeDtypeStruct((n, h), table.dtype),
        mesh=mesh,
        scratch_types=[
            pltpu.VMEM((wpw, w), jnp.int32),
            pltpu.VMEM((w, h), jnp.float32),
            pltpu.VMEM((w, h), jnp.float32),
            pltpu.SemaphoreType.DMA,
            pltpu.SemaphoreType.DMA,
            pltpu.SemaphoreType.DMA,
            pltpu.SemaphoreType.DMA,
        ],
    )
    def gather_kernel(x_hbm, i_hbm, o_hbm, idxv, rows0, rows1, gs0, gs1, ws0, ws1):
        wid = lax.axis_index("subcore") * 2 + lax.axis_index("core")
        row0 = wid * wpw
        tok0 = wid * (wpw * w)
        pltpu.sync_copy(i_hbm.at[pl.ds(row0, wpw)], idxv)
        pltpu.async_copy(x_hbm.at[idxv.at[0]], rows0, gs0)
        pltpu.async_copy(x_hbm.at[idxv.at[1]], rows1, gs1)

        @pl.loop(0, half)
        def _(k):
            i = 2 * k
            dst0 = o_hbm.at[pl.ds(tok0 + i * w, w)]
            dst1 = o_hbm.at[pl.ds(tok0 + (i + 1) * w, w)]
            pltpu.make_async_copy(x_hbm.at[idxv.at[i]], rows0, gs0).wait()
            pltpu.async_copy(rows0, dst0, ws0)
            pltpu.make_async_copy(x_hbm.at[idxv.at[i + 1]], rows1, gs1).wait()
            pltpu.async_copy(rows1, dst1, ws1)

            @pl.when(k < half - 1)
            def _():
                pltpu.make_async_copy(rows0, dst0, ws0).wait()
                pltpu.async_copy(x_hbm.at[idxv.at[i + 2]], rows0, gs0)
                pltpu.make_async_copy(rows1, dst1, ws1).wait()
                pltpu.async_copy(x_hbm.at[idxv.at[i + 3]], rows1, gs1)

        tail = o_hbm.at[pl.ds(tok0, w)]
        pltpu.make_async_copy(rows0, tail, ws0).wait()
        pltpu.make_async_copy(rows1, tail, ws1).wait()

    return gather_kernel(table, idx2d)


def _ln_body(g_ref, t_ref, te_ref, ga_ref, be_ref, o_ref):
    h = g_ref.shape[1]
    x = g_ref[...]
    t = t_ref[0]
    te = te_ref[...]
    tv = jnp.where(t > 0.5, te[1][None, :], te[0][None, :])
    x = x + tv
    s1 = jnp.sum(x, axis=1, keepdims=True)
    s2 = jnp.sum(x * x, axis=1, keepdims=True)
    mu = s1 * (1.0 / h)
    var = jnp.maximum(s2 * (1.0 / h) - mu * mu, 0.0)
    rstd = lax.rsqrt(var + _LN_EPS)
    o_ref[...] = (x - mu) * rstd * ga_ref[...][None, :] + be_ref[...][None, :]


def _tc_add_ln_chunk(big, gathered, tt3, type_emb, gamma, beta, n, block0):
    nc, h = gathered.shape
    nb = nc // _BT

    def body(b_ref, g_ref, t_ref, te_ref, ga_ref, be_ref, o_ref):
        del b_ref
        _ln_body(g_ref, t_ref, te_ref, ga_ref, be_ref, o_ref)

    def body0(g_ref, t_ref, te_ref, ga_ref, be_ref, o_ref):
        _ln_body(g_ref, t_ref, te_ref, ga_ref, be_ref, o_ref)

    data_specs = [
        pl.BlockSpec((_BT, h), lambda i: (i, 0)),
        pl.BlockSpec((1, _BT, 1), lambda i: (i, 0, 0)),
        pl.BlockSpec((2, h), lambda i: (0, 0)),
        pl.BlockSpec((h,), lambda i: (0,)),
        pl.BlockSpec((h,), lambda i: (0,)),
    ]
    out_spec = pl.BlockSpec((_BT, h), lambda i: (block0 + i, 0))
    out_shape = jax.ShapeDtypeStruct((n, h), jnp.float32)
    if big is None:
        return pl.pallas_call(
            body0,
            grid=(nb,),
            in_specs=data_specs,
            out_specs=out_spec,
            out_shape=out_shape,
        )(gathered, tt3, type_emb, gamma, beta)
    return pl.pallas_call(
        body,
        grid=(nb,),
        in_specs=[pl.BlockSpec(memory_space=pl.ANY)] + data_specs,
        out_specs=out_spec,
        out_shape=out_shape,
        input_output_aliases={0: 0},
    )(big, gathered, tt3, type_emb, gamma, beta)


def kernel(input_ids, token_type_ids, word_emb, type_emb, gamma, beta):
    b, l = input_ids.shape
    h = word_emb.shape[1]
    n = b * l
    ids = input_ids.reshape(n // _GATHER_WINDOW, _GATHER_WINDOW).astype(jnp.int32)
    tt3 = token_type_ids.reshape(n // _BT, _BT, 1).astype(jnp.float32)
    gathered = _sc_gather(word_emb, ids)
    out = _tc_add_ln_chunk(None, gathered, tt3, type_emb, gamma, beta, n, 0)
    return out.reshape(b, l, h)


# speedup vs baseline: 1.0608x; 1.0608x over previous
import functools

import jax
import jax.numpy as jnp
from jax import lax
from jax.experimental import pallas as pl
from jax.experimental.pallas import tpu as pltpu
from jax.experimental.pallas import tpu_sc as plsc

_LN_EPS = 1e-12
_GATHER_WINDOW = 128
_NUM_CHUNKS = 1
_BT = 8192


def _sc_gather(table, idx2d):
    nwin, w = idx2d.shape
    n = nwin * w
    h = table.shape[1]
    nworkers = 32
    wpw = nwin // nworkers
    half = wpw // 2
    mesh = plsc.VectorSubcoreMesh(core_axis_name="core", subcore_axis_name="subcore")

    @functools.partial(
        pl.kernel,
        out_type=jax.ShapeDtypeStruct((n, h), table.dtype),
        mesh=mesh,
        scratch_types=[
            pltpu.VMEM((wpw, w), jnp.int32),
            pltpu.VMEM((w, h), jnp.float32),
            pltpu.VMEM((w, h), jnp.float32),
            pltpu.SemaphoreType.DMA,
            pltpu.SemaphoreType.DMA,
            pltpu.SemaphoreType.DMA,
            pltpu.SemaphoreType.DMA,
        ],
    )
    def gather_kernel(x_hbm, i_hbm, o_hbm, idxv, rows0, rows1, gs0, gs1, ws0, ws1):
        wid = lax.axis_index("subcore") * 2 + lax.axis_index("core")
        row0 = wid * wpw
        tok0 = wid * (wpw * w)
        pltpu.sync_copy(i_hbm.at[pl.ds(row0, wpw)], idxv)
        pltpu.async_copy(x_hbm.at[idxv.at[0]], rows0, gs0)
        pltpu.async_copy(x_hbm.at[idxv.at[1]], rows1, gs1)

        @pl.loop(0, half)
        def _(k):
            i = 2 * k
            dst0 = o_hbm.at[pl.ds(tok0 + i * w, w)]
            dst1 = o_hbm.at[pl.ds(tok0 + (i + 1) * w, w)]
            pltpu.make_async_copy(x_hbm.at[idxv.at[i]], rows0, gs0).wait()
            pltpu.async_copy(rows0, dst0, ws0)
            pltpu.make_async_copy(x_hbm.at[idxv.at[i + 1]], rows1, gs1).wait()
            pltpu.async_copy(rows1, dst1, ws1)

            @pl.when(k < half - 1)
            def _():
                pltpu.make_async_copy(rows0, dst0, ws0).wait()
                pltpu.async_copy(x_hbm.at[idxv.at[i + 2]], rows0, gs0)
                pltpu.make_async_copy(rows1, dst1, ws1).wait()
                pltpu.async_copy(x_hbm.at[idxv.at[i + 3]], rows1, gs1)

        tail = o_hbm.at[pl.ds(tok0, w)]
        pltpu.make_async_copy(rows0, tail, ws0).wait()
        pltpu.make_async_copy(rows1, tail, ws1).wait()

    return gather_kernel(table, idx2d)


def _ln_body(g_ref, t_ref, te_ref, ga_ref, be_ref, o_ref):
    h = g_ref.shape[1]
    x = g_ref[...]
    t = t_ref[0]
    te = te_ref[...]
    tv = jnp.where(t > 0.5, te[1][None, :], te[0][None, :])
    x = x + tv
    s1 = jnp.sum(x, axis=1, keepdims=True)
    s2 = jnp.sum(x * x, axis=1, keepdims=True)
    mu = s1 * (1.0 / h)
    var = jnp.maximum(s2 * (1.0 / h) - mu * mu, 0.0)
    rstd = lax.rsqrt(var + _LN_EPS)
    o_ref[...] = (x - mu) * rstd * ga_ref[...][None, :] + be_ref[...][None, :]


def _tc_add_ln_chunk(big, gathered, tt3, type_emb, gamma, beta, n, block0):
    nc, h = gathered.shape
    nb = nc // _BT

    def body(b_ref, g_ref, t_ref, te_ref, ga_ref, be_ref, o_ref):
        del b_ref
        _ln_body(g_ref, t_ref, te_ref, ga_ref, be_ref, o_ref)

    def body0(g_ref, t_ref, te_ref, ga_ref, be_ref, o_ref):
        _ln_body(g_ref, t_ref, te_ref, ga_ref, be_ref, o_ref)

    data_specs = [
        pl.BlockSpec((_BT, h), lambda i: (i, 0)),
        pl.BlockSpec((1, _BT, 1), lambda i: (i, 0, 0)),
        pl.BlockSpec((2, h), lambda i: (0, 0)),
        pl.BlockSpec((h,), lambda i: (0,)),
        pl.BlockSpec((h,), lambda i: (0,)),
    ]
    out_spec = pl.BlockSpec((_BT, h), lambda i: (block0 + i, 0))
    out_shape = jax.ShapeDtypeStruct((n, h), jnp.float32)
    if big is None:
        return pl.pallas_call(
            body0,
            grid=(nb,),
            in_specs=data_specs,
            out_specs=out_spec,
            out_shape=out_shape,
        )(gathered, tt3, type_emb, gamma, beta)
    return pl.pallas_call(
        body,
        grid=(nb,),
        in_specs=[pl.BlockSpec(memory_space=pl.ANY)] + data_specs,
        out_specs=out_spec,
        out_shape=out_shape,
        input_output_aliases={0: 0},
    )(big, gathered, tt3, type_emb, gamma, beta)


def kernel(input_ids, token_type_ids, word_emb, type_emb, gamma, beta):
    b, l = input_ids.shape
    h = word_emb.shape[1]
    n = b * l
    ids = input_ids.reshape(n // _GATHER_WINDOW, _GATHER_WINDOW).astype(jnp.int32)
    tt3 = token_type_ids.reshape(n // _BT, _BT, 1).astype(jnp.float32)
    gathered = _sc_gather(word_emb, ids)
    out = _tc_add_ln_chunk(None, gathered, tt3, type_emb, gamma, beta, n, 0)
    return out.reshape(b, l, h)


# speedup vs baseline: 1.0709x; 1.0096x over previous
import functools

import jax
import jax.numpy as jnp
from jax import lax
from jax.experimental import pallas as pl
from jax.experimental.pallas import tpu as pltpu
from jax.experimental.pallas import tpu_sc as plsc

_LN_EPS = 1e-12
_GATHER_WINDOW = 128
_NUM_CHUNKS = 1
_BT = 10240


def _sc_gather(table, idx2d):
    nwin, w = idx2d.shape
    n = nwin * w
    h = table.shape[1]
    nworkers = 32
    wpw = nwin // nworkers
    half = wpw // 2
    mesh = plsc.VectorSubcoreMesh(core_axis_name="core", subcore_axis_name="subcore")

    @functools.partial(
        pl.kernel,
        out_type=jax.ShapeDtypeStruct((n, h), table.dtype),
        mesh=mesh,
        scratch_types=[
            pltpu.VMEM((wpw, w), jnp.int32),
            pltpu.VMEM((w, h), jnp.float32),
            pltpu.VMEM((w, h), jnp.float32),
            pltpu.SemaphoreType.DMA,
            pltpu.SemaphoreType.DMA,
            pltpu.SemaphoreType.DMA,
            pltpu.SemaphoreType.DMA,
        ],
    )
    def gather_kernel(x_hbm, i_hbm, o_hbm, idxv, rows0, rows1, gs0, gs1, ws0, ws1):
        wid = lax.axis_index("subcore") * 2 + lax.axis_index("core")
        row0 = wid * wpw
        tok0 = wid * (wpw * w)
        pltpu.sync_copy(i_hbm.at[pl.ds(row0, wpw)], idxv)
        pltpu.async_copy(x_hbm.at[idxv.at[0]], rows0, gs0)
        pltpu.async_copy(x_hbm.at[idxv.at[1]], rows1, gs1)

        @pl.loop(0, half)
        def _(k):
            i = 2 * k
            dst0 = o_hbm.at[pl.ds(tok0 + i * w, w)]
            dst1 = o_hbm.at[pl.ds(tok0 + (i + 1) * w, w)]
            pltpu.make_async_copy(x_hbm.at[idxv.at[i]], rows0, gs0).wait()
            pltpu.async_copy(rows0, dst0, ws0)
            pltpu.make_async_copy(x_hbm.at[idxv.at[i + 1]], rows1, gs1).wait()
            pltpu.async_copy(rows1, dst1, ws1)

            @pl.when(k < half - 1)
            def _():
                pltpu.make_async_copy(rows0, dst0, ws0).wait()
                pltpu.async_copy(x_hbm.at[idxv.at[i + 2]], rows0, gs0)
                pltpu.make_async_copy(rows1, dst1, ws1).wait()
                pltpu.async_copy(x_hbm.at[idxv.at[i + 3]], rows1, gs1)

        tail = o_hbm.at[pl.ds(tok0, w)]
        pltpu.make_async_copy(rows0, tail, ws0).wait()
        pltpu.make_async_copy(rows1, tail, ws1).wait()

    return gather_kernel(table, idx2d)


def _ln_body(g_ref, t_ref, te_ref, ga_ref, be_ref, o_ref):
    h = g_ref.shape[1]
    x = g_ref[...]
    t = t_ref[0]
    te = te_ref[...]
    tv = jnp.where(t > 0.5, te[1][None, :], te[0][None, :])
    x = x + tv
    s1 = jnp.sum(x, axis=1, keepdims=True)
    s2 = jnp.sum(x * x, axis=1, keepdims=True)
    mu = s1 * (1.0 / h)
    var = jnp.maximum(s2 * (1.0 / h) - mu * mu, 0.0)
    rstd = lax.rsqrt(var + _LN_EPS)
    o_ref[...] = (x - mu) * rstd * ga_ref[...][None, :] + be_ref[...][None, :]


def _tc_add_ln_chunk(big, gathered, tt3, type_emb, gamma, beta, n, block0):
    nc, h = gathered.shape
    nb = nc // _BT

    def body(b_ref, g_ref, t_ref, te_ref, ga_ref, be_ref, o_ref):
        del b_ref
        _ln_body(g_ref, t_ref, te_ref, ga_ref, be_ref, o_ref)

    def body0(g_ref, t_ref, te_ref, ga_ref, be_ref, o_ref):
        _ln_body(g_ref, t_ref, te_ref, ga_ref, be_ref, o_ref)

    data_specs = [
        pl.BlockSpec((_BT, h), lambda i: (i, 0)),
        pl.BlockSpec((1, _BT, 1), lambda i: (i, 0, 0)),
        pl.BlockSpec((2, h), lambda i: (0, 0)),
        pl.BlockSpec((h,), lambda i: (0,)),
        pl.BlockSpec((h,), lambda i: (0,)),
    ]
    out_spec = pl.BlockSpec((_BT, h), lambda i: (block0 + i, 0))
    out_shape = jax.ShapeDtypeStruct((n, h), jnp.float32)
    if big is None:
        return pl.pallas_call(
            body0,
            grid=(nb,),
            in_specs=data_specs,
            out_specs=out_spec,
            out_shape=out_shape,
        )(gathered, tt3, type_emb, gamma, beta)
    return pl.pallas_call(
        body,
        grid=(nb,),
        in_specs=[pl.BlockSpec(memory_space=pl.ANY)] + data_specs,
        out_specs=out_spec,
        out_shape=out_shape,
        input_output_aliases={0: 0},
    )(big, gathered, tt3, type_emb, gamma, beta)


def kernel(input_ids, token_type_ids, word_emb, type_emb, gamma, beta):
    b, l = input_ids.shape
    h = word_emb.shape[1]
    n = b * l
    ids = input_ids.reshape(n // _GATHER_WINDOW, _GATHER_WINDOW).astype(jnp.int32)
    tt3 = token_type_ids.reshape(n // _BT, _BT, 1).astype(jnp.float32)
    gathered = _sc_gather(word_emb, ids)
    out = _tc_add_ln_chunk(None, gathered, tt3, type_emb, gamma, beta, n, 0)
    return out.reshape(b, l, h)
